# baseline (speedup 1.0000x reference)
import functools

import jax
import jax.numpy as jnp
from jax import lax
from jax.experimental import pallas as pl
from jax.experimental.pallas import tpu as pltpu
from jax.experimental.pallas import tpu_sc as plsc

B = 8
ATOMS_PER = 1250
EDGES_PER = 20000
ANGLES_PER = 40000
N_ATOMS = B * ATOMS_PER
N_EDGES = B * EDGES_PER
N_ANGLES = B * ANGLES_PER
FEATURE_DIM = 128
ANGLE_DIM = 16
MAX_N = 4
CUTOFF = 5.0
TB_CUTOFF = 4.0
EPS = 1e-8

_SC_MESH = dict(core_axis_name="c", subcore_axis_name="s")
NUM_SC = 2
NUM_TILES = 32


def _swish(x):
    return x * jax.nn.sigmoid(x)


@functools.partial(jax.jit, static_argnames=("window",))
def _sc_gather(table, idx, window=128):
    m = idx.shape[0]
    n, d = table.shape
    idx2 = idx.reshape(1, m)
    mesh = plsc.VectorSubcoreMesh(**_SC_MESH)

    @functools.partial(
        pl.kernel,
        mesh=mesh,
        out_type=jax.ShapeDtypeStruct((m, d), table.dtype),
        compiler_params=pltpu.CompilerParams(use_tc_tiling_on_sc=False),
    )
    def k(x_hbm, i_hbm, o_hbm):
        def body(i_vmem, o_vmem):
            pltpu.sync_copy(x_hbm.at[i_vmem.at[0]], o_vmem)

        pltpu.emit_pipeline(
            body,
            grid=(m // window,),
            in_specs=[pl.BlockSpec((1, window), index_map=lambda i: (0, i))],
            out_specs=[pl.BlockSpec((window, d), index_map=lambda i: (i, 0))],
            core_axis_name=("c", "s"),
            dimension_semantics=(pltpu.PARALLEL,),
        )(i_hbm, o_hbm)

    return k(table, idx2)


@functools.partial(jax.jit, static_argnames=("w",))
def _sc_scatter_add(rows, idx, zeros, w):
    m, d = rows.shape
    ch = 128
    per_tile = m // NUM_TILES
    n_full = per_tile // ch
    rem = per_tile - n_full * ch
    wslice = w // 16
    mesh = plsc.VectorSubcoreMesh(**_SC_MESH)
    scratch = [
        pltpu.VMEM((ch,), jnp.int32),
        pltpu.VMEM((ch, d), jnp.float32),
        pltpu.VMEM_SHARED((w, d), jnp.float32),
    ]
    if rem:
        scratch.append(pltpu.VMEM((rem,), jnp.int32))

    @functools.partial(
        pl.kernel,
        mesh=mesh,
        out_type=jax.ShapeDtypeStruct((NUM_SC, w, d), jnp.float32),
        scratch_types=scratch,
        compiler_params=pltpu.CompilerParams(use_tc_tiling_on_sc=False),
    )
    def k(rows_hbm, idx_hbm, z_hbm, out_hbm, idxv, rowsv, acc, *maybe_idxr):
        c = lax.axis_index("c")
        s = lax.axis_index("s")
        pltpu.sync_copy(z_hbm, acc.at[pl.ds(s * wslice, wslice)])
        plsc.subcore_barrier()
        base = (c * 16 + s) * per_tile

        @pl.loop(0, n_full)
        def _(i):
            off = base + i * ch
            pltpu.sync_copy(idx_hbm.at[pl.ds(off, ch)], idxv)
            pltpu.sync_copy(rows_hbm.at[pl.ds(off, ch)], rowsv)
            pltpu.sync_copy(rowsv, acc.at[idxv], add=True)

        if rem:
            idxr = maybe_idxr[0]
            off = base + n_full * ch
            pltpu.sync_copy(idx_hbm.at[pl.ds(off, rem)], idxr)
            pltpu.sync_copy(rows_hbm.at[pl.ds(off, rem)], rowsv.at[pl.ds(0, rem)])
            pltpu.sync_copy(rowsv.at[pl.ds(0, rem)], acc.at[idxr], add=True)
        plsc.subcore_barrier()
        pltpu.sync_copy(
            acc.at[pl.ds(s * wslice, wslice)],
            out_hbm.at[c, pl.ds(s * wslice, wslice)],
        )

    return k(rows, idx, zeros)


@jax.jit
def _sc_dstik(dst, ikl):
    m = N_ANGLES
    per_tile = m // NUM_TILES
    slab = 2000
    n_slab = per_tile // slab
    mesh = plsc.VectorSubcoreMesh(**_SC_MESH)

    @functools.partial(
        pl.kernel,
        mesh=mesh,
        out_type=jax.ShapeDtypeStruct((m,), jnp.int32),
        scratch_types=[
            pltpu.VMEM((EDGES_PER,), jnp.int32),
            pltpu.VMEM((slab,), jnp.int32),
            pltpu.VMEM((slab,), jnp.int32),
        ],
        compiler_params=pltpu.CompilerParams(
            use_tc_tiling_on_sc=False, needs_layout_passes=False
        ),
    )
    def k(dst_hbm, ikl_hbm, out_hbm, dslice, ikv, ov):
        c = lax.axis_index("c")
        s = lax.axis_index("s")
        base = (c * 16 + s) * per_tile
        boff = pl.multiple_of((base // ANGLES_PER) * EDGES_PER, 8)
        pltpu.sync_copy(dst_hbm.at[pl.ds(boff, EDGES_PER)], dslice)

        @pl.loop(0, n_slab)
        def _(i):
            off = base + i * slab
            pltpu.sync_copy(ikl_hbm.at[pl.ds(off, slab)], ikv)

            @pl.loop(0, slab // 16)
            def _(j):
                idx = ikv[pl.ds(j * 16, 16)]
                ov[pl.ds(j * 16, 16)] = plsc.load_gather(dslice, [idx])

            pltpu.sync_copy(ov, out_hbm.at[pl.ds(off, slab)])

    return k(dst, ikl)


@functools.partial(jax.jit, static_argnames=("w_win",))
def _sc_angle_stage(gate, ik, ij_loc, wbasis, zeros, w_win):
    m = ik.shape[0]
    d = gate.shape[1]
    ch = 128
    per_tile = m // NUM_TILES
    n_full = per_tile // ch
    rem = per_tile - n_full * ch
    wslice = w_win // 16
    mesh = plsc.VectorSubcoreMesh(**_SC_MESH)
    scratch = [
        pltpu.VMEM((ch,), jnp.int32),
        pltpu.VMEM((ch,), jnp.int32),
        pltpu.VMEM((ch, d), jnp.float32),
        pltpu.VMEM((ch, d), jnp.float32),
        pltpu.VMEM_SHARED((w_win, d), jnp.float32),
        pltpu.SemaphoreType.DMA,
    ]
    if rem:
        scratch.append(pltpu.VMEM((rem,), jnp.int32))
        scratch.append(pltpu.VMEM((rem,), jnp.int32))

    @functools.partial(
        pl.kernel,
        mesh=mesh,
        out_type=jax.ShapeDtypeStruct((NUM_SC, w_win, d), jnp.float32),
        scratch_types=scratch,
        compiler_params=pltpu.CompilerParams(use_tc_tiling_on_sc=False),
    )
    def k(gate_hbm, ik_hbm, ij_hbm, w_hbm, z_hbm, out_hbm,
          ikv, ijv, gbuf, wbuf, acc, sem, *rest):
        c = lax.axis_index("c")
        s = lax.axis_index("s")
        pltpu.sync_copy(z_hbm, acc.at[pl.ds(s * wslice, wslice)])
        plsc.subcore_barrier()
        base = (c * 16 + s) * per_tile

        def do_chunk(off, nn, ikr, ijr):
            pltpu.sync_copy(ik_hbm.at[pl.ds(off, nn)], ikr)
            pltpu.async_copy(gate_hbm.at[ikr], gbuf.at[pl.ds(0, nn)], sem).wait()
            pltpu.sync_copy(w_hbm.at[pl.ds(off, nn)], wbuf.at[pl.ds(0, nn)])

            @pl.loop(0, nn)
            def _(r):
                wbuf[r, :] = wbuf[r, :] * gbuf[r, :]

            pltpu.sync_copy(ij_hbm.at[pl.ds(off, nn)], ijr)
            pltpu.sync_copy(wbuf.at[pl.ds(0, nn)], acc.at[ijr], add=True)

        @pl.loop(0, n_full)
        def _(i):
            do_chunk(base + i * ch, ch, ikv, ijv)

        if rem:
            do_chunk(base + n_full * ch, rem, rest[0], rest[1])
        plsc.subcore_barrier()
        pltpu.sync_copy(
            acc.at[pl.ds(s * wslice, wslice)],
            out_hbm.at[c, pl.ds(s * wslice, wslice)],
        )

    return k(gate, ik, ij_loc, wbasis, zeros)


_EB_GEOM = 2000


def _lane_iota():
    return jax.lax.broadcasted_iota(jnp.int32, (16, 16), 1)


def _row_iota():
    return jax.lax.broadcasted_iota(jnp.int32, (16, 16), 0)


def _tc_edge_geom_body(ps_ref, pd_ref, eoff_ref, cell_ref, encw_ref, encb_ref,
                       egeo_ref, ef0_ref, ef_ref):
    b = (pl.program_id(0) * _EB_GEOM) // EDGES_PER
    row = _row_iota()
    lane = _lane_iota()
    cmat = jnp.zeros((16, 16), jnp.float32)
    for i in range(3):
        for j in range(3):
            sel = ((row == i) & (lane == j)).astype(jnp.float32)
            cmat = cmat + cell_ref[b, 3 * i + j] * sel
    eb = _EB_GEOM
    lanes1 = jax.lax.broadcasted_iota(jnp.int32, (1, 16), 1)
    mask3 = (lanes1 < 3).astype(jnp.float32)
    off = jnp.dot(eoff_ref[...], cmat, preferred_element_type=jnp.float32)
    vec = (pd_ref[...] - ps_ref[...]) * mask3 + off
    sel3 = ((row < 3)).astype(jnp.float32)
    d2 = jnp.dot(vec * vec, sel3, preferred_element_type=jnp.float32)
    dist = jnp.sqrt(d2)
    inv = 1.0 / (dist + EPS)
    scale = jnp.sqrt(2.0 / CUTOFF)
    nvec = jnp.where(lanes1 < MAX_N, lanes1 + 1, 0).astype(jnp.float32)
    ef0 = scale * jnp.sin((jnp.pi / CUTOFF) * dist * nvec) * inv
    lane3 = (lanes1 == 3).astype(jnp.float32)
    egeo_ref[...] = vec + dist * lane3
    ef0_ref[...] = ef0
    ef_ref[...] = _swish(
        jnp.dot(ef0, encw_ref[...], preferred_element_type=jnp.float32)
        + encb_ref[...]
    )


def _tc_edge_geom(ps, pd, eoff16, cell9, encw16, enc_b):
    grid = (N_EDGES // _EB_GEOM,)
    return pl.pallas_call(
        _tc_edge_geom_body,
        grid=grid,
        in_specs=[
            pl.BlockSpec((_EB_GEOM, 16), lambda i: (i, 0)),
            pl.BlockSpec((_EB_GEOM, 16), lambda i: (i, 0)),
            pl.BlockSpec((_EB_GEOM, 16), lambda i: (i, 0)),
            pl.BlockSpec(memory_space=pltpu.SMEM),
            pl.BlockSpec((16, FEATURE_DIM), lambda i: (0, 0)),
            pl.BlockSpec((1, FEATURE_DIM), lambda i: (0, 0)),
        ],
        out_specs=[
            pl.BlockSpec((_EB_GEOM, 16), lambda i: (i, 0)),
            pl.BlockSpec((_EB_GEOM, 16), lambda i: (i, 0)),
            pl.BlockSpec((_EB_GEOM, FEATURE_DIM), lambda i: (i, 0)),
        ],
        out_shape=[
            jax.ShapeDtypeStruct((N_EDGES, 16), jnp.float32),
            jax.ShapeDtypeStruct((N_EDGES, 16), jnp.float32),
            jax.ShapeDtypeStruct((N_EDGES, FEATURE_DIM), jnp.float32),
        ],
    )(ps, pd, eoff16, cell9, encw16, enc_b)


_AB_GEOM = 4000


def _tc_angle_geom_body(gij_ref, gik_ref, w_ref):
    row = _row_iota()
    sel3 = (row < 3).astype(jnp.float32)
    selr3 = (row == 3).astype(jnp.float32)
    gij = gij_ref[...]
    gik = gik_ref[...]
    dot = jnp.dot(gij * gik, sel3, preferred_element_type=jnp.float32)
    dij = jnp.dot(gij, selr3, preferred_element_type=jnp.float32)
    dik = jnp.dot(gik, selr3, preferred_element_type=jnp.float32)
    cos = dot / (dij * dik + EPS)
    inv = 1.0 / (dik + EPS)
    lanes1 = jax.lax.broadcasted_iota(jnp.int32, (1, 16), 1)
    nlane = (lanes1 // 4 + 1).astype(jnp.float32)
    g = jnp.sin((jnp.pi / TB_CUTOFF) * dik * nlane) * inv
    x = jnp.clip(dik / TB_CUTOFF, 0.0, 1.0)
    fc = 1.0 - 6.0 * x**5 + 15.0 * x**4 - 10.0 * x**3
    lsel = lanes1 % 4
    m0 = (lsel == 0).astype(jnp.float32)
    m1 = (lsel == 1).astype(jnp.float32)
    m2 = (lsel == 2).astype(jnp.float32)
    m3 = (lsel == 3).astype(jnp.float32)
    p_all = (
        m0
        + cos * m1
        + (0.5 * (3.0 * cos**2 - 1.0)) * m2
        + (0.5 * (5.0 * cos**3 - 3.0 * cos)) * m3
    )
    w_ref[...] = g * p_all * fc


def _tc_angle_geom(gij, gik):
    grid = (N_ANGLES // _AB_GEOM,)
    return pl.pallas_call(
        _tc_angle_geom_body,
        grid=grid,
        in_specs=[
            pl.BlockSpec((_AB_GEOM, 16), lambda i: (i, 0)),
            pl.BlockSpec((_AB_GEOM, 16), lambda i: (i, 0)),
        ],
        out_specs=pl.BlockSpec((_AB_GEOM, 16), lambda i: (i, 0)),
        out_shape=jax.ShapeDtypeStruct((N_ANGLES, 16), jnp.float32),
    )(gij, gik)


_EB_GATE = 4000


def _tc_gate_body(aj_ref, w_ref, out_ref):
    out_ref[...] = jax.nn.sigmoid(
        jnp.dot(aj_ref[...], w_ref[...], preferred_element_type=jnp.float32)
    )


def _tc_gate(aj, gate_W):
    n, _ = aj.shape
    eb = 2000
    grid = (n // eb,)
    return pl.pallas_call(
        _tc_gate_body,
        grid=grid,
        in_specs=[
            pl.BlockSpec((eb, FEATURE_DIM), lambda i: (i, 0)),
            pl.BlockSpec((FEATURE_DIM, ANGLE_DIM), lambda i: (0, 0)),
        ],
        out_specs=pl.BlockSpec((eb, ANGLE_DIM), lambda i: (i, 0)),
        out_shape=jax.ShapeDtypeStruct((n, ANGLE_DIM), jnp.float32),
    )(aj, gate_W)


_AB_MSG = 8000


def _tc_msg_body(w_ref, g_ref, out_ref):
    out_ref[...] = w_ref[...] * g_ref[...]


def _tc_msg(w, gik_gate):
    grid = (N_ANGLES // _AB_MSG,)
    return pl.pallas_call(
        _tc_msg_body,
        grid=grid,
        in_specs=[
            pl.BlockSpec((_AB_MSG, 16), lambda i: (i, 0)),
            pl.BlockSpec((_AB_MSG, 16), lambda i: (i, 0)),
        ],
        out_specs=pl.BlockSpec((_AB_MSG, 16), lambda i: (i, 0)),
        out_shape=jax.ShapeDtypeStruct((N_ANGLES, 16), jnp.float32),
    )(w, gik_gate)


_EB_BLK = 1600


def _tc_edge_block_body(ai_ref, aj_ref, ef_ref, eagg_ref, ef0_ref,
                        wcat_ref, tbw_ref, tbb_ref, ew0_ref, aw0_ref,
                        efo_ref, amsg_ref):
    ef2 = ef_ref[...] + _swish(
        jnp.dot(eagg_ref[...], tbw_ref[...], preferred_element_type=jnp.float32)
        + tbb_ref[...]
    )
    feat = jnp.concatenate(
        [ai_ref[...], aj_ref[...], ef2.astype(jnp.bfloat16)], axis=1
    )
    mm = jnp.dot(feat, wcat_ref[...], preferred_element_type=jnp.float32)
    ew = jnp.dot(ef0_ref[...], ew0_ref[...], preferred_element_type=jnp.float32)
    aw = jnp.dot(ef0_ref[...], aw0_ref[...], preferred_element_type=jnp.float32)
    d = FEATURE_DIM
    efo_ref[...] = ef2 + _swish(mm[:, :d]) * jax.nn.sigmoid(mm[:, d : 2 * d]) * ew
    amsg_ref[...] = (
        _swish(mm[:, 2 * d : 3 * d]) * jax.nn.sigmoid(mm[:, 3 * d :]) * aw
    )


def _tc_edge_block(ai, aj, ef, eagg, ef0, wcat, tbw, tbb, ew0, aw0):
    grid = (N_EDGES // _EB_BLK,)
    d = FEATURE_DIM
    return pl.pallas_call(
        _tc_edge_block_body,
        grid=grid,
        in_specs=[
            pl.BlockSpec((_EB_BLK, d), lambda i: (i, 0)),
            pl.BlockSpec((_EB_BLK, d), lambda i: (i, 0)),
            pl.BlockSpec((_EB_BLK, d), lambda i: (i, 0)),
            pl.BlockSpec((_EB_BLK, ANGLE_DIM), lambda i: (i, 0)),
            pl.BlockSpec((_EB_BLK, 16), lambda i: (i, 0)),
            pl.BlockSpec((3 * d, 4 * d), lambda i: (0, 0)),
            pl.BlockSpec((ANGLE_DIM, d), lambda i: (0, 0)),
            pl.BlockSpec((1, d), lambda i: (0, 0)),
            pl.BlockSpec((16, d), lambda i: (0, 0)),
            pl.BlockSpec((16, d), lambda i: (0, 0)),
        ],
        out_specs=[
            pl.BlockSpec((_EB_BLK, d), lambda i: (i, 0)),
            pl.BlockSpec((_EB_BLK, d), lambda i: (i, 0)),
        ],
        out_shape=[
            jax.ShapeDtypeStruct((N_EDGES, d), jnp.float32),
            jax.ShapeDtypeStruct((N_EDGES, d), jnp.float32),
        ],
    )(ai, aj, ef, eagg, ef0, wcat, tbw, tbb, ew0, aw0)


_ATB = 2000


def _tc_add3_body(a_ref, p0_ref, p1_ref, out_ref, outh_ref):
    v = a_ref[...] + p0_ref[...] + p1_ref[...]
    out_ref[...] = v
    outh_ref[...] = v.astype(jnp.bfloat16)


def _tc_add3(a, p0, p1):
    grid = (N_ATOMS // _ATB,)
    d = FEATURE_DIM
    spec = pl.BlockSpec((_ATB, d), lambda i: (i, 0))
    return pl.pallas_call(
        _tc_add3_body,
        grid=grid,
        in_specs=[spec, spec, spec],
        out_specs=[spec, spec],
        out_shape=[
            jax.ShapeDtypeStruct((N_ATOMS, d), jnp.float32),
            jax.ShapeDtypeStruct((N_ATOMS, d), jnp.bfloat16),
        ],
    )(a, p0, p1)


def _tc_cast_body(a_ref, outh_ref):
    outh_ref[...] = a_ref[...].astype(jnp.bfloat16)


def _tc_cast_bf16(a):
    grid = (N_ATOMS // _ATB,)
    d = FEATURE_DIM
    spec = pl.BlockSpec((_ATB, d), lambda i: (i, 0))
    return pl.pallas_call(
        _tc_cast_body,
        grid=grid,
        in_specs=[spec],
        out_specs=spec,
        out_shape=jax.ShapeDtypeStruct((N_ATOMS, d), jnp.bfloat16),
    )(a)


def _tc_final_body(af_ref, w1_ref, b1_ref, w2_ref, b2_ref, w3r_ref, b3_ref,
                   out_ref):
    h = _swish(
        jnp.dot(af_ref[0], w1_ref[...], preferred_element_type=jnp.float32)
        + b1_ref[...]
    )
    h = _swish(
        jnp.dot(h, w2_ref[...], preferred_element_type=jnp.float32)
        + b2_ref[...]
    )
    s = jnp.sum(h * w3r_ref[...]) + ATOMS_PER * b3_ref[0, 0]
    out_ref[...] = jnp.broadcast_to(s, (1, 1, FEATURE_DIM))


def _tc_final(atom_f, w1, b1, w2, b2, w3r, b3):
    d = FEATURE_DIM
    af3 = atom_f.reshape(B, ATOMS_PER, d)
    return pl.pallas_call(
        _tc_final_body,
        grid=(B,),
        in_specs=[
            pl.BlockSpec((1, ATOMS_PER, d), lambda i: (i, 0, 0)),
            pl.BlockSpec((d, d), lambda i: (0, 0)),
            pl.BlockSpec((1, d), lambda i: (0, 0)),
            pl.BlockSpec((d, d), lambda i: (0, 0)),
            pl.BlockSpec((1, d), lambda i: (0, 0)),
            pl.BlockSpec((1, d), lambda i: (0, 0)),
            pl.BlockSpec(memory_space=pltpu.SMEM),
        ],
        out_specs=pl.BlockSpec((1, 1, FEATURE_DIM), lambda i: (i, 0, 0)),
        out_shape=jax.ShapeDtypeStruct((B, 1, FEATURE_DIM), jnp.float32),
    )(af3, w1, b1, w2, b2, w3r, b3)


def kernel(atomic_numbers, pos, edge_index, edge_offsets, cell, three_body_indices, total_num_atoms, total_num_edges, total_num_angles, embed_table, enc_W, enc_b, tb_gate_W_0, tb_edge_W_0, tb_edge_b_0, e_phi_W_0, e_sig_W_0, e_w0_0, a_phi_W_0, a_sig_W_0, a_w0_0, tb_gate_W_1, tb_edge_W_1, tb_edge_b_1, e_phi_W_1, e_sig_W_1, e_w0_1, a_phi_W_1, a_sig_W_1, a_w0_1, tb_gate_W_2, tb_edge_W_2, tb_edge_b_2, e_phi_W_2, e_sig_W_2, e_w0_2, a_phi_W_2, a_sig_W_2, a_w0_2, tb_gate_W_3, tb_edge_W_3, tb_edge_b_3, e_phi_W_3, e_sig_W_3, e_w0_3, a_phi_W_3, a_sig_W_3, a_w0_3, en_W1, en_b1, en_W2, en_b2, en_W3, en_b3):
    blocks = [
        (tb_gate_W_0, tb_edge_W_0, tb_edge_b_0, e_phi_W_0, e_sig_W_0, e_w0_0, a_phi_W_0, a_sig_W_0, a_w0_0),
        (tb_gate_W_1, tb_edge_W_1, tb_edge_b_1, e_phi_W_1, e_sig_W_1, e_w0_1, a_phi_W_1, a_sig_W_1, a_w0_1),
        (tb_gate_W_2, tb_edge_W_2, tb_edge_b_2, e_phi_W_2, e_sig_W_2, e_w0_2, a_phi_W_2, a_sig_W_2, a_w0_2),
        (tb_gate_W_3, tb_edge_W_3, tb_edge_b_3, e_phi_W_3, e_sig_W_3, e_w0_3, a_phi_W_3, a_sig_W_3, a_w0_3),
    ]

    src = edge_index[0].astype(jnp.int32)
    dst = edge_index[1].astype(jnp.int32)
    ab = (jnp.arange(N_ANGLES, dtype=jnp.int32) // ANGLES_PER)
    tbi = three_body_indices.astype(jnp.int32)
    ij_glob = tbi[:, 0] + ab * EDGES_PER
    ik_glob = tbi[:, 1] + ab * EDGES_PER
    ij_loc = tbi[:, 0] + (ab % (B // 2)) * EDGES_PER
    pos16 = jnp.pad(pos, ((0, 0), (0, 13)))
    cell9 = cell.reshape(B, 9)
    an_pad = jnp.pad(atomic_numbers.astype(jnp.int32), (0, 240))
    zeros16 = jnp.zeros((EDGES_PER * (B // 2) // 16, ANGLE_DIM), jnp.float32)
    zeros128 = jnp.zeros((N_ATOMS // 16, FEATURE_DIM), jnp.float32)
    enc_b2 = enc_b.reshape(1, FEATURE_DIM)
    eoff16 = jnp.pad(edge_offsets, ((0, 0), (0, 13)))
    encw16 = jnp.pad(enc_W, ((0, 12), (0, 0)))

    psd = _sc_gather(pos16, jnp.concatenate([src, dst]))
    ps = psd[:N_EDGES]
    pd = psd[N_EDGES:]
    egeo, ef0, edge_f = _tc_edge_geom(ps, pd, eoff16, cell9, encw16, enc_b2)
    gcat = _sc_gather(egeo, jnp.concatenate([ij_glob, ik_glob]))
    gij = gcat[:N_ANGLES]
    gik = gcat[N_ANGLES:]
    w = _tc_angle_geom(gij, gik)

    atom_f = _sc_gather(embed_table, an_pad)[:N_ATOMS]
    atom_fh = _tc_cast_bf16(atom_f)

    dstsrc = jnp.concatenate([dst, src])
    dstik = _sc_dstik(dst, tbi[:, 1])
    for (gate_W, tbw, tbb, e_phi, e_sig, e_w0, a_phi, a_sig, a_w0) in blocks:
        wcat = jnp.concatenate(
            [e_phi, e_sig, a_phi, a_sig], axis=1
        ).astype(jnp.bfloat16)
        ajai = _sc_gather(atom_fh, dstsrc)
        aj = ajai[:N_EDGES]
        ai = ajai[N_EDGES:]
        gate_atom = _tc_gate(atom_f, gate_W)
        gate_ik = _sc_gather(gate_atom, dstik)
        msg = _tc_msg(w, gate_ik)
        eagg = _sc_scatter_add(msg, ij_loc, zeros16, EDGES_PER * (B // 2))
        eagg = eagg.reshape(N_EDGES, ANGLE_DIM)
        edge_f, amsg = _tc_edge_block(
            ai, aj, edge_f, eagg, ef0, wcat, tbw,
            tbb.reshape(1, FEATURE_DIM),
            jnp.pad(e_w0, ((0, 12), (0, 0))),
            jnp.pad(a_w0, ((0, 12), (0, 0))),
        )
        parts = _sc_scatter_add(amsg, src, zeros128, N_ATOMS)
        atom_f, atom_fh = _tc_add3(atom_f, parts[0], parts[1])

    out = _tc_final(
        atom_f, en_W1, en_b1.reshape(1, FEATURE_DIM),
        en_W2, en_b2.reshape(1, FEATURE_DIM),
        en_W3.reshape(1, FEATURE_DIM), en_b3.reshape(1, 1),
    )
    return out[:, 0, 0]


# speedup vs baseline: 1.2953x; 1.2953x over previous
import functools

import jax
import jax.numpy as jnp
from jax import lax
from jax.experimental import pallas as pl
from jax.experimental.pallas import tpu as pltpu
from jax.experimental.pallas import tpu_sc as plsc

B = 8
ATOMS_PER = 1250
EDGES_PER = 20000
ANGLES_PER = 40000
N_ATOMS = B * ATOMS_PER
N_EDGES = B * EDGES_PER
N_ANGLES = B * ANGLES_PER
FEATURE_DIM = 128
ANGLE_DIM = 16
MAX_N = 4
CUTOFF = 5.0
TB_CUTOFF = 4.0
EPS = 1e-8

_SC_MESH = dict(core_axis_name="c", subcore_axis_name="s")
NUM_SC = 2
NUM_TILES = 32


def _swish(x):
    return x * jax.nn.sigmoid(x)


@functools.partial(jax.jit, static_argnames=("window",))
def _sc_gather(table, idx, window=128):
    m = idx.shape[0]
    n, d = table.shape
    idx2 = idx.reshape(1, m)
    mesh = plsc.VectorSubcoreMesh(**_SC_MESH)

    @functools.partial(
        pl.kernel,
        mesh=mesh,
        out_type=jax.ShapeDtypeStruct((m, d), table.dtype),
        compiler_params=pltpu.CompilerParams(use_tc_tiling_on_sc=False),
    )
    def k(x_hbm, i_hbm, o_hbm):
        def body(i_vmem, o_vmem):
            pltpu.sync_copy(x_hbm.at[i_vmem.at[0]], o_vmem)

        pltpu.emit_pipeline(
            body,
            grid=(m // window,),
            in_specs=[pl.BlockSpec((1, window), index_map=lambda i: (0, i))],
            out_specs=[pl.BlockSpec((window, d), index_map=lambda i: (i, 0))],
            core_axis_name=("c", "s"),
            dimension_semantics=(pltpu.PARALLEL,),
        )(i_hbm, o_hbm)

    return k(table, idx2)


@functools.partial(jax.jit, static_argnames=("w",))
def _sc_scatter_add(rows, idx, zeros, w):
    m, d = rows.shape
    ch = 128
    per_tile = m // NUM_TILES
    n_full = per_tile // ch
    rem = per_tile - n_full * ch
    wslice = w // 16
    mesh = plsc.VectorSubcoreMesh(**_SC_MESH)
    scratch = [
        pltpu.VMEM((ch,), jnp.int32),
        pltpu.VMEM((ch, d), jnp.float32),
        pltpu.VMEM_SHARED((w, d), jnp.float32),
    ]
    if rem:
        scratch.append(pltpu.VMEM((rem,), jnp.int32))

    @functools.partial(
        pl.kernel,
        mesh=mesh,
        out_type=jax.ShapeDtypeStruct((NUM_SC, w, d), jnp.float32),
        scratch_types=scratch,
        compiler_params=pltpu.CompilerParams(use_tc_tiling_on_sc=False),
    )
    def k(rows_hbm, idx_hbm, z_hbm, out_hbm, idxv, rowsv, acc, *maybe_idxr):
        c = lax.axis_index("c")
        s = lax.axis_index("s")
        pltpu.sync_copy(z_hbm, acc.at[pl.ds(s * wslice, wslice)])
        plsc.subcore_barrier()
        base = (c * 16 + s) * per_tile

        @pl.loop(0, n_full)
        def _(i):
            off = base + i * ch
            pltpu.sync_copy(idx_hbm.at[pl.ds(off, ch)], idxv)
            pltpu.sync_copy(rows_hbm.at[pl.ds(off, ch)], rowsv)
            pltpu.sync_copy(rowsv, acc.at[idxv], add=True)

        if rem:
            idxr = maybe_idxr[0]
            off = base + n_full * ch
            pltpu.sync_copy(idx_hbm.at[pl.ds(off, rem)], idxr)
            pltpu.sync_copy(rows_hbm.at[pl.ds(off, rem)], rowsv.at[pl.ds(0, rem)])
            pltpu.sync_copy(rowsv.at[pl.ds(0, rem)], acc.at[idxr], add=True)
        plsc.subcore_barrier()
        pltpu.sync_copy(
            acc.at[pl.ds(s * wslice, wslice)],
            out_hbm.at[c, pl.ds(s * wslice, wslice)],
        )

    return k(rows, idx, zeros)


@jax.jit
def _sc_dstik(dst, ikl):
    m = N_ANGLES
    per_tile = m // NUM_TILES
    slab = 2000
    n_slab = per_tile // slab
    mesh = plsc.VectorSubcoreMesh(**_SC_MESH)

    @functools.partial(
        pl.kernel,
        mesh=mesh,
        out_type=jax.ShapeDtypeStruct((m,), jnp.int32),
        scratch_types=[
            pltpu.VMEM((EDGES_PER,), jnp.int32),
            pltpu.VMEM((slab,), jnp.int32),
            pltpu.VMEM((slab,), jnp.int32),
        ],
        compiler_params=pltpu.CompilerParams(
            use_tc_tiling_on_sc=False, needs_layout_passes=False
        ),
    )
    def k(dst_hbm, ikl_hbm, out_hbm, dslice, ikv, ov):
        c = lax.axis_index("c")
        s = lax.axis_index("s")
        base = (c * 16 + s) * per_tile
        boff = pl.multiple_of((base // ANGLES_PER) * EDGES_PER, 8)
        pltpu.sync_copy(dst_hbm.at[pl.ds(boff, EDGES_PER)], dslice)

        @pl.loop(0, n_slab)
        def _(i):
            off = base + i * slab
            pltpu.sync_copy(ikl_hbm.at[pl.ds(off, slab)], ikv)

            @pl.loop(0, slab // 16)
            def _(j):
                idx = ikv[pl.ds(j * 16, 16)]
                ov[pl.ds(j * 16, 16)] = plsc.load_gather(dslice, [idx])

            pltpu.sync_copy(ov, out_hbm.at[pl.ds(off, slab)])

    return k(dst, ikl)


@functools.partial(jax.jit, static_argnames=("w_win",))
def _sc_angle_stage(gate, ik, ij_loc, wbasis, zeros, w_win):
    m = ik.shape[0]
    d = gate.shape[1]
    ch = 128
    per_tile = m // NUM_TILES
    n_full = per_tile // ch
    rem = per_tile - n_full * ch
    wslice = w_win // 16
    mesh = plsc.VectorSubcoreMesh(**_SC_MESH)
    scratch = [
        pltpu.VMEM((ch,), jnp.int32),
        pltpu.VMEM((ch,), jnp.int32),
        pltpu.VMEM((ch, d), jnp.float32),
        pltpu.VMEM((ch, d), jnp.float32),
        pltpu.VMEM_SHARED((w_win, d), jnp.float32),
        pltpu.SemaphoreType.DMA,
    ]
    if rem:
        scratch.append(pltpu.VMEM((rem,), jnp.int32))
        scratch.append(pltpu.VMEM((rem,), jnp.int32))

    @functools.partial(
        pl.kernel,
        mesh=mesh,
        out_type=jax.ShapeDtypeStruct((NUM_SC, w_win, d), jnp.float32),
        scratch_types=scratch,
        compiler_params=pltpu.CompilerParams(use_tc_tiling_on_sc=False),
    )
    def k(gate_hbm, ik_hbm, ij_hbm, w_hbm, z_hbm, out_hbm,
          ikv, ijv, gbuf, wbuf, acc, sem, *rest):
        c = lax.axis_index("c")
        s = lax.axis_index("s")
        pltpu.sync_copy(z_hbm, acc.at[pl.ds(s * wslice, wslice)])
        plsc.subcore_barrier()
        base = (c * 16 + s) * per_tile

        def do_chunk(off, nn, ikr, ijr):
            pltpu.sync_copy(ik_hbm.at[pl.ds(off, nn)], ikr)
            pltpu.async_copy(gate_hbm.at[ikr], gbuf.at[pl.ds(0, nn)], sem).wait()
            pltpu.sync_copy(w_hbm.at[pl.ds(off, nn)], wbuf.at[pl.ds(0, nn)])

            @pl.loop(0, nn)
            def _(r):
                wbuf[r, :] = wbuf[r, :] * gbuf[r, :]

            pltpu.sync_copy(ij_hbm.at[pl.ds(off, nn)], ijr)
            pltpu.sync_copy(wbuf.at[pl.ds(0, nn)], acc.at[ijr], add=True)

        @pl.loop(0, n_full)
        def _(i):
            do_chunk(base + i * ch, ch, ikv, ijv)

        if rem:
            do_chunk(base + n_full * ch, rem, rest[0], rest[1])
        plsc.subcore_barrier()
        pltpu.sync_copy(
            acc.at[pl.ds(s * wslice, wslice)],
            out_hbm.at[c, pl.ds(s * wslice, wslice)],
        )

    return k(gate, ik, ij_loc, wbasis, zeros)


_EB_GEOM = 2000


def _lane_iota():
    return jax.lax.broadcasted_iota(jnp.int32, (16, 16), 1)


def _row_iota():
    return jax.lax.broadcasted_iota(jnp.int32, (16, 16), 0)


_PACK = 8
_NEP = N_EDGES // _PACK
_NAP = N_ANGLES // _PACK
_EBP_GEOM = 2000


def _iota2(shape, dim):
    return jax.lax.broadcasted_iota(jnp.int32, shape, dim)


def _tc_edge_geom_body(ps_ref, pd_ref, eoff_ref, cell_ref, egeo_ref,
                       ef0_ref):
    pid = pl.program_id(0)
    row0 = pid * _EBP_GEOM
    ba = (row0 * _PACK) // EDGES_PER
    bb = ((row0 + _EBP_GEOM) * _PACK - 1) // EDGES_PER
    row = _iota2((128, 128), 0)
    lane = _iota2((128, 128), 1)
    grp = (row // 16 == lane // 16)
    ri = row % 16
    li = lane % 16
    cma = jnp.zeros((128, 128), jnp.float32)
    cmb = jnp.zeros((128, 128), jnp.float32)
    for i in range(3):
        for j in range(3):
            sel = (grp & (ri == i) & (li == j)).astype(jnp.float32)
            cma = cma + cell_ref[ba, 3 * i + j] * sel
            cmb = cmb + cell_ref[bb, 3 * i + j] * sel
    offa = jnp.dot(eoff_ref[...], cma, preferred_element_type=jnp.float32)
    offb = jnp.dot(eoff_ref[...], cmb, preferred_element_type=jnp.float32)
    rowg = _iota2((_EBP_GEOM, 1), 0) + row0
    ina = (rowg * _PACK // EDGES_PER == ba).astype(jnp.float32)
    off = offa * ina + offb * (1.0 - ina)
    lanes1 = _iota2((1, 128), 1) % 16
    mask3 = (lanes1 < 3).astype(jnp.float32)
    vec = (pd_ref[...] - ps_ref[...]) * mask3 + off
    sel3 = (grp & (ri < 3)).astype(jnp.float32)
    d2 = jnp.dot(vec * vec, sel3, preferred_element_type=jnp.float32)
    dist = jnp.sqrt(d2)
    inv = 1.0 / (dist + EPS)
    scale = jnp.sqrt(2.0 / CUTOFF)
    nvec = jnp.where(lanes1 < MAX_N, lanes1 + 1, 0).astype(jnp.float32)
    ef0 = scale * jnp.sin((jnp.pi / CUTOFF) * dist * nvec) * inv
    lane3 = (lanes1 == 3).astype(jnp.float32)
    egeo_ref[...] = vec + dist * lane3
    ef0_ref[...] = ef0


def _tc_edge_geom(psp, pdp, eoffp, cell9):
    grid = (_NEP // _EBP_GEOM,)
    spec = pl.BlockSpec((_EBP_GEOM, 128), lambda i: (i, 0))
    return pl.pallas_call(
        _tc_edge_geom_body,
        grid=grid,
        in_specs=[
            spec,
            spec,
            spec,
            pl.BlockSpec(memory_space=pltpu.SMEM),
        ],
        out_specs=[spec, spec],
        out_shape=[
            jax.ShapeDtypeStruct((_NEP, 128), jnp.float32),
            jax.ShapeDtypeStruct((_NEP, 128), jnp.float32),
        ],
    )(psp, pdp, eoffp, cell9)


def _tc_edge_enc_body(ef0p_ref, encbig_ref, encb8_ref, ef_ref):
    ef_ref[...] = _swish(
        jnp.dot(ef0p_ref[...], encbig_ref[...],
                preferred_element_type=jnp.float32)
        + encb8_ref[...]
    )


def _tc_edge_enc(ef0p, encbig, encb8):
    eb = 1000
    grid = (_NEP // eb,)
    return pl.pallas_call(
        _tc_edge_enc_body,
        grid=grid,
        in_specs=[
            pl.BlockSpec((eb, 128), lambda i: (i, 0)),
            pl.BlockSpec((128, _PACK * FEATURE_DIM), lambda i: (0, 0)),
            pl.BlockSpec((1, _PACK * FEATURE_DIM), lambda i: (0, 0)),
        ],
        out_specs=pl.BlockSpec((eb, _PACK * FEATURE_DIM), lambda i: (i, 0)),
        out_shape=jax.ShapeDtypeStruct((_NEP, _PACK * FEATURE_DIM),
                                       jnp.float32),
    )(ef0p, encbig, encb8)


_AB_GEOM = 4000


def _tc_angle_geom_body(gij_ref, gik_ref, w_ref):
    row = _iota2((128, 128), 0)
    lane = _iota2((128, 128), 1)
    grp = (row // 16 == lane // 16)
    sel3 = (grp & (row % 16 < 3)).astype(jnp.float32)
    selr3 = (grp & (row % 16 == 3)).astype(jnp.float32)
    gij = gij_ref[...]
    gik = gik_ref[...]
    dot = jnp.dot(gij * gik, sel3, preferred_element_type=jnp.float32)
    dij = jnp.dot(gij, selr3, preferred_element_type=jnp.float32)
    dik = jnp.dot(gik, selr3, preferred_element_type=jnp.float32)
    cos = dot / (dij * dik + EPS)
    inv = 1.0 / (dik + EPS)
    lanes1 = _iota2((1, 128), 1) % 16
    nlane = (lanes1 // 4 + 1).astype(jnp.float32)
    g = jnp.sin((jnp.pi / TB_CUTOFF) * dik * nlane) * inv
    x = jnp.clip(dik / TB_CUTOFF, 0.0, 1.0)
    fc = 1.0 - 6.0 * x**5 + 15.0 * x**4 - 10.0 * x**3
    lsel = lanes1 % 4
    m0 = (lsel == 0).astype(jnp.float32)
    m1 = (lsel == 1).astype(jnp.float32)
    m2 = (lsel == 2).astype(jnp.float32)
    m3 = (lsel == 3).astype(jnp.float32)
    p_all = (
        m0
        + cos * m1
        + (0.5 * (3.0 * cos**2 - 1.0)) * m2
        + (0.5 * (5.0 * cos**3 - 3.0 * cos)) * m3
    )
    w_ref[...] = g * p_all * fc


def _tc_angle_geom(gijp, gikp):
    eb = 2000
    grid = (_NAP // eb,)
    spec = pl.BlockSpec((eb, 128), lambda i: (i, 0))
    return pl.pallas_call(
        _tc_angle_geom_body,
        grid=grid,
        in_specs=[spec, spec],
        out_specs=spec,
        out_shape=jax.ShapeDtypeStruct((_NAP, 128), jnp.float32),
    )(gijp, gikp)


_EB_GATE = 4000


def _tc_gate_body(aj_ref, w_ref, out_ref):
    out_ref[...] = jax.nn.sigmoid(
        jnp.dot(aj_ref[...], w_ref[...], preferred_element_type=jnp.float32)
    )


def _tc_gate(aj, gate_W):
    n, _ = aj.shape
    eb = 2000
    grid = (n // eb,)
    return pl.pallas_call(
        _tc_gate_body,
        grid=grid,
        in_specs=[
            pl.BlockSpec((eb, FEATURE_DIM), lambda i: (i, 0)),
            pl.BlockSpec((FEATURE_DIM, ANGLE_DIM), lambda i: (0, 0)),
        ],
        out_specs=pl.BlockSpec((eb, ANGLE_DIM), lambda i: (i, 0)),
        out_shape=jax.ShapeDtypeStruct((n, ANGLE_DIM), jnp.float32),
    )(aj, gate_W)


_AB_MSG = 8000


def _tc_msg_body(w_ref, g_ref, out_ref):
    out_ref[...] = w_ref[...] * g_ref[...]


def _tc_msg(wp, gik_gate_p):
    eb = 4000
    grid = (_NAP // eb,)
    spec = pl.BlockSpec((eb, 128), lambda i: (i, 0))
    return pl.pallas_call(
        _tc_msg_body,
        grid=grid,
        in_specs=[spec, spec],
        out_specs=spec,
        out_shape=jax.ShapeDtypeStruct((_NAP, 128), jnp.float32),
    )(wp, gik_gate_p)


_EB_BLK = 1600


def _tc_edge_block_body(ai_ref, aj_ref, ef_ref, eagg_ref, ef0_ref,
                        wcat_ref, tbw_ref, tbb_ref, ew0_ref, aw0_ref,
                        efo_ref, amsg_ref):
    ef2 = ef_ref[...] + _swish(
        jnp.dot(eagg_ref[...], tbw_ref[...], preferred_element_type=jnp.float32)
        + tbb_ref[...]
    )
    feat = jnp.concatenate(
        [ai_ref[...], aj_ref[...], ef2.astype(jnp.bfloat16)], axis=1
    )
    mm = jnp.dot(feat, wcat_ref[...], preferred_element_type=jnp.float32)
    ew = jnp.dot(ef0_ref[...], ew0_ref[...], preferred_element_type=jnp.float32)
    aw = jnp.dot(ef0_ref[...], aw0_ref[...], preferred_element_type=jnp.float32)
    d = FEATURE_DIM
    efo_ref[...] = ef2 + _swish(mm[:, :d]) * jax.nn.sigmoid(mm[:, d : 2 * d]) * ew
    amsg_ref[...] = (
        _swish(mm[:, 2 * d : 3 * d]) * jax.nn.sigmoid(mm[:, 3 * d :]) * aw
    )


def _tc_edge_block(ai, aj, ef, eagg, ef0, wcat, tbw, tbb, ew0, aw0):
    grid = (N_EDGES // _EB_BLK,)
    d = FEATURE_DIM
    return pl.pallas_call(
        _tc_edge_block_body,
        grid=grid,
        in_specs=[
            pl.BlockSpec((_EB_BLK, d), lambda i: (i, 0)),
            pl.BlockSpec((_EB_BLK, d), lambda i: (i, 0)),
            pl.BlockSpec((_EB_BLK, d), lambda i: (i, 0)),
            pl.BlockSpec((_EB_BLK, ANGLE_DIM), lambda i: (i, 0)),
            pl.BlockSpec((_EB_BLK, 16), lambda i: (i, 0)),
            pl.BlockSpec((3 * d, 4 * d), lambda i: (0, 0)),
            pl.BlockSpec((ANGLE_DIM, d), lambda i: (0, 0)),
            pl.BlockSpec((1, d), lambda i: (0, 0)),
            pl.BlockSpec((16, d), lambda i: (0, 0)),
            pl.BlockSpec((16, d), lambda i: (0, 0)),
        ],
        out_specs=[
            pl.BlockSpec((_EB_BLK, d), lambda i: (i, 0)),
            pl.BlockSpec((_EB_BLK, d), lambda i: (i, 0)),
        ],
        out_shape=[
            jax.ShapeDtypeStruct((N_EDGES, d), jnp.float32),
            jax.ShapeDtypeStruct((N_EDGES, d), jnp.float32),
        ],
    )(ai, aj, ef, eagg, ef0, wcat, tbw, tbb, ew0, aw0)


_ATB = 2000


def _tc_add3_body(a_ref, p0_ref, p1_ref, out_ref, outh_ref):
    v = a_ref[...] + p0_ref[...] + p1_ref[...]
    out_ref[...] = v
    outh_ref[...] = v.astype(jnp.bfloat16)


def _tc_add3(a, p0, p1):
    grid = (N_ATOMS // _ATB,)
    d = FEATURE_DIM
    spec = pl.BlockSpec((_ATB, d), lambda i: (i, 0))
    return pl.pallas_call(
        _tc_add3_body,
        grid=grid,
        in_specs=[spec, spec, spec],
        out_specs=[spec, spec],
        out_shape=[
            jax.ShapeDtypeStruct((N_ATOMS, d), jnp.float32),
            jax.ShapeDtypeStruct((N_ATOMS, d), jnp.bfloat16),
        ],
    )(a, p0, p1)


def _tc_cast_body(a_ref, outh_ref):
    outh_ref[...] = a_ref[...].astype(jnp.bfloat16)


def _tc_cast_bf16(a):
    grid = (N_ATOMS // _ATB,)
    d = FEATURE_DIM
    spec = pl.BlockSpec((_ATB, d), lambda i: (i, 0))
    return pl.pallas_call(
        _tc_cast_body,
        grid=grid,
        in_specs=[spec],
        out_specs=spec,
        out_shape=jax.ShapeDtypeStruct((N_ATOMS, d), jnp.bfloat16),
    )(a)


def _tc_final_body(af_ref, w1_ref, b1_ref, w2_ref, b2_ref, w3r_ref, b3_ref,
                   out_ref):
    h = _swish(
        jnp.dot(af_ref[0], w1_ref[...], preferred_element_type=jnp.float32)
        + b1_ref[...]
    )
    h = _swish(
        jnp.dot(h, w2_ref[...], preferred_element_type=jnp.float32)
        + b2_ref[...]
    )
    s = jnp.sum(h * w3r_ref[...]) + ATOMS_PER * b3_ref[0, 0]
    out_ref[...] = jnp.broadcast_to(s, (1, 1, FEATURE_DIM))


def _tc_final(atom_f, w1, b1, w2, b2, w3r, b3):
    d = FEATURE_DIM
    af3 = atom_f.reshape(B, ATOMS_PER, d)
    return pl.pallas_call(
        _tc_final_body,
        grid=(B,),
        in_specs=[
            pl.BlockSpec((1, ATOMS_PER, d), lambda i: (i, 0, 0)),
            pl.BlockSpec((d, d), lambda i: (0, 0)),
            pl.BlockSpec((1, d), lambda i: (0, 0)),
            pl.BlockSpec((d, d), lambda i: (0, 0)),
            pl.BlockSpec((1, d), lambda i: (0, 0)),
            pl.BlockSpec((1, d), lambda i: (0, 0)),
            pl.BlockSpec(memory_space=pltpu.SMEM),
        ],
        out_specs=pl.BlockSpec((1, 1, FEATURE_DIM), lambda i: (i, 0, 0)),
        out_shape=jax.ShapeDtypeStruct((B, 1, FEATURE_DIM), jnp.float32),
    )(af3, w1, b1, w2, b2, w3r, b3)


def kernel(atomic_numbers, pos, edge_index, edge_offsets, cell, three_body_indices, total_num_atoms, total_num_edges, total_num_angles, embed_table, enc_W, enc_b, tb_gate_W_0, tb_edge_W_0, tb_edge_b_0, e_phi_W_0, e_sig_W_0, e_w0_0, a_phi_W_0, a_sig_W_0, a_w0_0, tb_gate_W_1, tb_edge_W_1, tb_edge_b_1, e_phi_W_1, e_sig_W_1, e_w0_1, a_phi_W_1, a_sig_W_1, a_w0_1, tb_gate_W_2, tb_edge_W_2, tb_edge_b_2, e_phi_W_2, e_sig_W_2, e_w0_2, a_phi_W_2, a_sig_W_2, a_w0_2, tb_gate_W_3, tb_edge_W_3, tb_edge_b_3, e_phi_W_3, e_sig_W_3, e_w0_3, a_phi_W_3, a_sig_W_3, a_w0_3, en_W1, en_b1, en_W2, en_b2, en_W3, en_b3):
    blocks = [
        (tb_gate_W_0, tb_edge_W_0, tb_edge_b_0, e_phi_W_0, e_sig_W_0, e_w0_0, a_phi_W_0, a_sig_W_0, a_w0_0),
        (tb_gate_W_1, tb_edge_W_1, tb_edge_b_1, e_phi_W_1, e_sig_W_1, e_w0_1, a_phi_W_1, a_sig_W_1, a_w0_1),
        (tb_gate_W_2, tb_edge_W_2, tb_edge_b_2, e_phi_W_2, e_sig_W_2, e_w0_2, a_phi_W_2, a_sig_W_2, a_w0_2),
        (tb_gate_W_3, tb_edge_W_3, tb_edge_b_3, e_phi_W_3, e_sig_W_3, e_w0_3, a_phi_W_3, a_sig_W_3, a_w0_3),
    ]

    src = edge_index[0].astype(jnp.int32)
    dst = edge_index[1].astype(jnp.int32)
    ab = (jnp.arange(N_ANGLES, dtype=jnp.int32) // ANGLES_PER)
    tbi = three_body_indices.astype(jnp.int32)
    ij_glob = tbi[:, 0] + ab * EDGES_PER
    ik_glob = tbi[:, 1] + ab * EDGES_PER
    ij_loc = tbi[:, 0] + (ab % (B // 2)) * EDGES_PER
    pos16 = jnp.pad(pos, ((0, 0), (0, 13)))
    cell9 = cell.reshape(B, 9)
    an_pad = jnp.pad(atomic_numbers.astype(jnp.int32), (0, 240))
    zeros16 = jnp.zeros((EDGES_PER * (B // 2) // 16, ANGLE_DIM), jnp.float32)
    zeros128 = jnp.zeros((N_ATOMS // 16, FEATURE_DIM), jnp.float32)
    enc_b2 = enc_b.reshape(1, FEATURE_DIM)
    eoffp = jnp.pad(edge_offsets, ((0, 0), (0, 13))).reshape(_NEP, 128)
    encw16 = jnp.pad(enc_W, ((0, 12), (0, 0)))
    encbig = jnp.kron(jnp.eye(_PACK, dtype=jnp.float32), encw16)
    encb8 = jnp.tile(enc_b.reshape(1, FEATURE_DIM), (1, _PACK))

    psd = _sc_gather(pos16, jnp.concatenate([src, dst]))
    psp = psd[:N_EDGES].reshape(_NEP, 128)
    pdp = psd[N_EDGES:].reshape(_NEP, 128)
    egeop, ef0p = _tc_edge_geom(psp, pdp, eoffp, cell9)
    edge_f = _tc_edge_enc(ef0p, encbig, encb8).reshape(N_EDGES, FEATURE_DIM)
    egeo = egeop.reshape(N_EDGES, 16)
    ef0 = ef0p.reshape(N_EDGES, 16)
    gcat = _sc_gather(egeo, jnp.concatenate([ij_glob, ik_glob]))
    gijp = gcat[:N_ANGLES].reshape(_NAP, 128)
    gikp = gcat[N_ANGLES:].reshape(_NAP, 128)
    wp = _tc_angle_geom(gijp, gikp)

    atom_f = _sc_gather(embed_table, an_pad)[:N_ATOMS]
    atom_fh = _tc_cast_bf16(atom_f)

    dstsrc = jnp.concatenate([dst, src])
    dstik = _sc_dstik(dst, tbi[:, 1])
    for (gate_W, tbw, tbb, e_phi, e_sig, e_w0, a_phi, a_sig, a_w0) in blocks:
        wcat = jnp.concatenate(
            [e_phi, e_sig, a_phi, a_sig], axis=1
        ).astype(jnp.bfloat16)
        ajai = _sc_gather(atom_fh, dstsrc)
        aj = ajai[:N_EDGES]
        ai = ajai[N_EDGES:]
        gate_atom = _tc_gate(atom_f, gate_W)
        gate_ik = _sc_gather(gate_atom, dstik)
        msg = _tc_msg(wp, gate_ik.reshape(_NAP, 128)).reshape(N_ANGLES, 16)
        eagg = _sc_scatter_add(msg, ij_loc, zeros16, EDGES_PER * (B // 2))
        eagg = eagg.reshape(N_EDGES, ANGLE_DIM)
        edge_f, amsg = _tc_edge_block(
            ai, aj, edge_f, eagg, ef0, wcat, tbw,
            tbb.reshape(1, FEATURE_DIM),
            jnp.pad(e_w0, ((0, 12), (0, 0))),
            jnp.pad(a_w0, ((0, 12), (0, 0))),
        )
        parts = _sc_scatter_add(amsg, src, zeros128, N_ATOMS)
        atom_f, atom_fh = _tc_add3(atom_f, parts[0], parts[1])

    out = _tc_final(
        atom_f, en_W1, en_b1.reshape(1, FEATURE_DIM),
        en_W2, en_b2.reshape(1, FEATURE_DIM),
        en_W3.reshape(1, FEATURE_DIM), en_b3.reshape(1, 1),
    )
    return out[:, 0, 0]


# speedup vs baseline: 1.4814x; 1.1437x over previous
import functools

import jax
import jax.numpy as jnp
from jax import lax
from jax.experimental import pallas as pl
from jax.experimental.pallas import tpu as pltpu
from jax.experimental.pallas import tpu_sc as plsc

B = 8
ATOMS_PER = 1250
EDGES_PER = 20000
ANGLES_PER = 40000
N_ATOMS = B * ATOMS_PER
N_EDGES = B * EDGES_PER
N_ANGLES = B * ANGLES_PER
FEATURE_DIM = 128
ANGLE_DIM = 16
MAX_N = 4
CUTOFF = 5.0
TB_CUTOFF = 4.0
EPS = 1e-8

_SC_MESH = dict(core_axis_name="c", subcore_axis_name="s")
NUM_SC = 2
NUM_TILES = 32


def _swish(x):
    return x * jax.nn.sigmoid(x)


@functools.partial(jax.jit, static_argnames=("window",))
def _sc_gather(table, idx, window=128):
    m = idx.shape[0]
    n, d = table.shape
    idx2 = idx.reshape(1, m)
    mesh = plsc.VectorSubcoreMesh(**_SC_MESH)

    @functools.partial(
        pl.kernel,
        mesh=mesh,
        out_type=jax.ShapeDtypeStruct((m, d), table.dtype),
        compiler_params=pltpu.CompilerParams(use_tc_tiling_on_sc=False),
    )
    def k(x_hbm, i_hbm, o_hbm):
        def body(i_vmem, o_vmem):
            pltpu.sync_copy(x_hbm.at[i_vmem.at[0]], o_vmem)

        pltpu.emit_pipeline(
            body,
            grid=(m // window,),
            in_specs=[pl.BlockSpec((1, window), index_map=lambda i: (0, i))],
            out_specs=[pl.BlockSpec((window, d), index_map=lambda i: (i, 0))],
            core_axis_name=("c", "s"),
            dimension_semantics=(pltpu.PARALLEL,),
        )(i_hbm, o_hbm)

    return k(table, idx2)


@functools.partial(jax.jit, static_argnames=("w",))
def _sc_scatter_add(rows, idx, zeros, w):
    m, d = rows.shape
    ch = 128
    per_tile = m // NUM_TILES
    n_full = per_tile // ch
    rem = per_tile - n_full * ch
    wslice = w // 16
    mesh = plsc.VectorSubcoreMesh(**_SC_MESH)
    scratch = [
        pltpu.VMEM((ch,), jnp.int32),
        pltpu.VMEM((ch, d), jnp.float32),
        pltpu.VMEM_SHARED((w, d), jnp.float32),
    ]
    if rem:
        scratch.append(pltpu.VMEM((rem,), jnp.int32))

    @functools.partial(
        pl.kernel,
        mesh=mesh,
        out_type=jax.ShapeDtypeStruct((NUM_SC, w, d), jnp.float32),
        scratch_types=scratch,
        compiler_params=pltpu.CompilerParams(use_tc_tiling_on_sc=False),
    )
    def k(rows_hbm, idx_hbm, z_hbm, out_hbm, idxv, rowsv, acc, *maybe_idxr):
        c = lax.axis_index("c")
        s = lax.axis_index("s")
        pltpu.sync_copy(z_hbm, acc.at[pl.ds(s * wslice, wslice)])
        plsc.subcore_barrier()
        base = (c * 16 + s) * per_tile

        @pl.loop(0, n_full)
        def _(i):
            off = base + i * ch
            pltpu.sync_copy(idx_hbm.at[pl.ds(off, ch)], idxv)
            pltpu.sync_copy(rows_hbm.at[pl.ds(off, ch)], rowsv)
            pltpu.sync_copy(rowsv, acc.at[idxv], add=True)

        if rem:
            idxr = maybe_idxr[0]
            off = base + n_full * ch
            pltpu.sync_copy(idx_hbm.at[pl.ds(off, rem)], idxr)
            pltpu.sync_copy(rows_hbm.at[pl.ds(off, rem)], rowsv.at[pl.ds(0, rem)])
            pltpu.sync_copy(rowsv.at[pl.ds(0, rem)], acc.at[idxr], add=True)
        plsc.subcore_barrier()
        pltpu.sync_copy(
            acc.at[pl.ds(s * wslice, wslice)],
            out_hbm.at[c, pl.ds(s * wslice, wslice)],
        )

    return k(rows, idx, zeros)


@jax.jit
def _sc_dstik(dst, ikl):
    m = N_ANGLES
    per_tile = m // NUM_TILES
    slab = 2000
    n_slab = per_tile // slab
    mesh = plsc.VectorSubcoreMesh(**_SC_MESH)

    @functools.partial(
        pl.kernel,
        mesh=mesh,
        out_type=jax.ShapeDtypeStruct((m,), jnp.int32),
        scratch_types=[
            pltpu.VMEM((EDGES_PER,), jnp.int32),
            pltpu.VMEM((slab,), jnp.int32),
            pltpu.VMEM((slab,), jnp.int32),
        ],
        compiler_params=pltpu.CompilerParams(
            use_tc_tiling_on_sc=False, needs_layout_passes=False
        ),
    )
    def k(dst_hbm, ikl_hbm, out_hbm, dslice, ikv, ov):
        c = lax.axis_index("c")
        s = lax.axis_index("s")
        base = (c * 16 + s) * per_tile
        boff = pl.multiple_of((base // ANGLES_PER) * EDGES_PER, 8)
        pltpu.sync_copy(dst_hbm.at[pl.ds(boff, EDGES_PER)], dslice)

        @pl.loop(0, n_slab)
        def _(i):
            off = base + i * slab
            pltpu.sync_copy(ikl_hbm.at[pl.ds(off, slab)], ikv)

            @pl.loop(0, slab // 16)
            def _(j):
                idx = ikv[pl.ds(j * 16, 16)]
                ov[pl.ds(j * 16, 16)] = plsc.load_gather(dslice, [idx])

            pltpu.sync_copy(ov, out_hbm.at[pl.ds(off, slab)])

    return k(dst, ikl)


@functools.partial(jax.jit, static_argnames=("w_win",))
def _sc_angle_stage(gate, ik, ij_loc, wbasis, zeros, w_win):
    m = ik.shape[0]
    d = gate.shape[1]
    ch = 128
    per_tile = m // NUM_TILES
    n_full = per_tile // ch
    rem = per_tile - n_full * ch
    wslice = w_win // 16
    mesh = plsc.VectorSubcoreMesh(**_SC_MESH)
    scratch = [
        pltpu.VMEM((ch,), jnp.int32),
        pltpu.VMEM((ch,), jnp.int32),
        pltpu.VMEM((ch, d), jnp.float32),
        pltpu.VMEM((ch, d), jnp.float32),
        pltpu.VMEM_SHARED((w_win, d), jnp.float32),
        pltpu.SemaphoreType.DMA,
    ]
    if rem:
        scratch.append(pltpu.VMEM((rem,), jnp.int32))
        scratch.append(pltpu.VMEM((rem,), jnp.int32))

    @functools.partial(
        pl.kernel,
        mesh=mesh,
        out_type=jax.ShapeDtypeStruct((NUM_SC, w_win, d), jnp.float32),
        scratch_types=scratch,
        compiler_params=pltpu.CompilerParams(use_tc_tiling_on_sc=False),
    )
    def k(gate_hbm, ik_hbm, ij_hbm, w_hbm, z_hbm, out_hbm,
          ikv, ijv, gbuf, wbuf, acc, sem, *rest):
        c = lax.axis_index("c")
        s = lax.axis_index("s")
        pltpu.sync_copy(z_hbm, acc.at[pl.ds(s * wslice, wslice)])
        plsc.subcore_barrier()
        base = (c * 16 + s) * per_tile

        def do_chunk(off, nn, ikr, ijr):
            pltpu.sync_copy(ik_hbm.at[pl.ds(off, nn)], ikr)
            pltpu.async_copy(gate_hbm.at[ikr], gbuf.at[pl.ds(0, nn)], sem).wait()
            pltpu.sync_copy(w_hbm.at[pl.ds(off, nn)], wbuf.at[pl.ds(0, nn)])

            @pl.loop(0, nn)
            def _(r):
                wbuf[r, :] = wbuf[r, :] * gbuf[r, :]

            pltpu.sync_copy(ij_hbm.at[pl.ds(off, nn)], ijr)
            pltpu.sync_copy(wbuf.at[pl.ds(0, nn)], acc.at[ijr], add=True)

        @pl.loop(0, n_full)
        def _(i):
            do_chunk(base + i * ch, ch, ikv, ijv)

        if rem:
            do_chunk(base + n_full * ch, rem, rest[0], rest[1])
        plsc.subcore_barrier()
        pltpu.sync_copy(
            acc.at[pl.ds(s * wslice, wslice)],
            out_hbm.at[c, pl.ds(s * wslice, wslice)],
        )

    return k(gate, ik, ij_loc, wbasis, zeros)


_EB_GEOM = 2000


def _lane_iota():
    return jax.lax.broadcasted_iota(jnp.int32, (16, 16), 1)


def _row_iota():
    return jax.lax.broadcasted_iota(jnp.int32, (16, 16), 0)


_PACK = 8
_NEP = N_EDGES // _PACK
_NAP = N_ANGLES // _PACK
_EBP_GEOM = 2000


def _iota2(shape, dim):
    return jax.lax.broadcasted_iota(jnp.int32, shape, dim)


def _tc_edge_geom_body(ps_ref, pd_ref, eoff_ref, cell_ref, egeo_ref,
                       ef0_ref):
    pid = pl.program_id(0)
    row0 = pid * _EBP_GEOM
    ba = (row0 * _PACK) // EDGES_PER
    bb = ((row0 + _EBP_GEOM) * _PACK - 1) // EDGES_PER
    row = _iota2((128, 128), 0)
    lane = _iota2((128, 128), 1)
    grp = (row // 16 == lane // 16)
    ri = row % 16
    li = lane % 16
    cma = jnp.zeros((128, 128), jnp.float32)
    cmb = jnp.zeros((128, 128), jnp.float32)
    for i in range(3):
        for j in range(3):
            sel = (grp & (ri == i) & (li == j)).astype(jnp.float32)
            cma = cma + cell_ref[ba, 3 * i + j] * sel
            cmb = cmb + cell_ref[bb, 3 * i + j] * sel
    offa = jnp.dot(eoff_ref[...], cma, preferred_element_type=jnp.float32)
    offb = jnp.dot(eoff_ref[...], cmb, preferred_element_type=jnp.float32)
    rowg = _iota2((_EBP_GEOM, 1), 0) + row0
    ina = (rowg * _PACK // EDGES_PER == ba).astype(jnp.float32)
    off = offa * ina + offb * (1.0 - ina)
    lanes1 = _iota2((1, 128), 1) % 16
    mask3 = (lanes1 < 3).astype(jnp.float32)
    vec = (pd_ref[...] - ps_ref[...]) * mask3 + off
    sel3 = (grp & (ri < 3)).astype(jnp.float32)
    d2 = jnp.dot(vec * vec, sel3, preferred_element_type=jnp.float32)
    dist = jnp.sqrt(d2)
    inv = 1.0 / (dist + EPS)
    scale = jnp.sqrt(2.0 / CUTOFF)
    nvec = jnp.where(lanes1 < MAX_N, lanes1 + 1, 0).astype(jnp.float32)
    ef0 = scale * jnp.sin((jnp.pi / CUTOFF) * dist * nvec) * inv
    lane3 = (lanes1 == 3).astype(jnp.float32)
    egeo_ref[...] = vec + dist * lane3
    ef0_ref[...] = ef0


def _tc_edge_geom(psp, pdp, eoffp, cell9):
    grid = (_NEP // _EBP_GEOM,)
    spec = pl.BlockSpec((_EBP_GEOM, 128), lambda i: (i, 0))
    return pl.pallas_call(
        _tc_edge_geom_body,
        grid=grid,
        in_specs=[
            spec,
            spec,
            spec,
            pl.BlockSpec(memory_space=pltpu.SMEM),
        ],
        out_specs=[spec, spec],
        out_shape=[
            jax.ShapeDtypeStruct((_NEP, 128), jnp.float32),
            jax.ShapeDtypeStruct((_NEP, 128), jnp.float32),
        ],
    )(psp, pdp, eoffp, cell9)


def _tc_edge_enc_body(ef0p_ref, encbig_ref, encb8_ref, ef_ref):
    ef_ref[...] = _swish(
        jnp.dot(ef0p_ref[...], encbig_ref[...],
                preferred_element_type=jnp.float32)
        + encb8_ref[...]
    )


def _tc_edge_enc(ef0p, encbig, encb8):
    eb = 1000
    grid = (_NEP // eb,)
    return pl.pallas_call(
        _tc_edge_enc_body,
        grid=grid,
        in_specs=[
            pl.BlockSpec((eb, 128), lambda i: (i, 0)),
            pl.BlockSpec((128, _PACK * FEATURE_DIM), lambda i: (0, 0)),
            pl.BlockSpec((1, _PACK * FEATURE_DIM), lambda i: (0, 0)),
        ],
        out_specs=pl.BlockSpec((eb, _PACK * FEATURE_DIM), lambda i: (i, 0)),
        out_shape=jax.ShapeDtypeStruct((_NEP, _PACK * FEATURE_DIM),
                                       jnp.float32),
    )(ef0p, encbig, encb8)


_AB_GEOM = 4000


def _tc_angle_geom_body(gij_ref, gik_ref, w_ref):
    row = _iota2((128, 128), 0)
    lane = _iota2((128, 128), 1)
    grp = (row // 16 == lane // 16)
    sel3 = (grp & (row % 16 < 3)).astype(jnp.float32)
    selr3 = (grp & (row % 16 == 3)).astype(jnp.float32)
    gij = gij_ref[...]
    gik = gik_ref[...]
    dot = jnp.dot(gij * gik, sel3, preferred_element_type=jnp.float32)
    dij = jnp.dot(gij, selr3, preferred_element_type=jnp.float32)
    dik = jnp.dot(gik, selr3, preferred_element_type=jnp.float32)
    cos = dot / (dij * dik + EPS)
    inv = 1.0 / (dik + EPS)
    lanes1 = _iota2((1, 128), 1) % 16
    nlane = (lanes1 // 4 + 1).astype(jnp.float32)
    g = jnp.sin((jnp.pi / TB_CUTOFF) * dik * nlane) * inv
    x = jnp.clip(dik / TB_CUTOFF, 0.0, 1.0)
    fc = 1.0 - 6.0 * x**5 + 15.0 * x**4 - 10.0 * x**3
    lsel = lanes1 % 4
    m0 = (lsel == 0).astype(jnp.float32)
    m1 = (lsel == 1).astype(jnp.float32)
    m2 = (lsel == 2).astype(jnp.float32)
    m3 = (lsel == 3).astype(jnp.float32)
    p_all = (
        m0
        + cos * m1
        + (0.5 * (3.0 * cos**2 - 1.0)) * m2
        + (0.5 * (5.0 * cos**3 - 3.0 * cos)) * m3
    )
    w_ref[...] = g * p_all * fc


def _tc_angle_geom(gijp, gikp):
    eb = 2000
    grid = (_NAP // eb,)
    spec = pl.BlockSpec((eb, 128), lambda i: (i, 0))
    return pl.pallas_call(
        _tc_angle_geom_body,
        grid=grid,
        in_specs=[spec, spec],
        out_specs=spec,
        out_shape=jax.ShapeDtypeStruct((_NAP, 128), jnp.float32),
    )(gijp, gikp)


_EB_GATE = 4000


def _tc_gate_body(aj_ref, w_ref, out_ref):
    out_ref[...] = jax.nn.sigmoid(
        jnp.dot(aj_ref[...], w_ref[...], preferred_element_type=jnp.float32)
    )


def _tc_gate(aj, gate_W):
    n, _ = aj.shape
    eb = 2000
    grid = (n // eb,)
    return pl.pallas_call(
        _tc_gate_body,
        grid=grid,
        in_specs=[
            pl.BlockSpec((eb, FEATURE_DIM), lambda i: (i, 0)),
            pl.BlockSpec((FEATURE_DIM, ANGLE_DIM), lambda i: (0, 0)),
        ],
        out_specs=pl.BlockSpec((eb, ANGLE_DIM), lambda i: (i, 0)),
        out_shape=jax.ShapeDtypeStruct((n, ANGLE_DIM), jnp.float32),
    )(aj, gate_W)


_AB_MSG = 8000


def _tc_msg_body(w_ref, g_ref, out_ref):
    out_ref[...] = w_ref[...] * g_ref[...]


def _tc_msg(wp, gik_gate_p):
    eb = 4000
    grid = (_NAP // eb,)
    spec = pl.BlockSpec((eb, 128), lambda i: (i, 0))
    return pl.pallas_call(
        _tc_msg_body,
        grid=grid,
        in_specs=[spec, spec],
        out_specs=spec,
        out_shape=jax.ShapeDtypeStruct((_NAP, 128), jnp.float32),
    )(wp, gik_gate_p)


_EB_BLK = 1600


def _tc_edge_block_body(ai_ref, aj_ref, ef_ref, eagg_ref, ef0_ref,
                        wcat_ref, tbw_ref, tbb_ref, ew0_ref, aw0_ref,
                        efo_ref, amsg_ref):
    ef2 = ef_ref[...] + _swish(
        jnp.dot(eagg_ref[...], tbw_ref[...], preferred_element_type=jnp.float32)
        + tbb_ref[...]
    )
    feat = jnp.concatenate([ai_ref[...], aj_ref[...], ef2], axis=1)
    mm = jnp.dot(feat, wcat_ref[...], preferred_element_type=jnp.float32)
    ew = jnp.dot(ef0_ref[...], ew0_ref[...], preferred_element_type=jnp.float32)
    aw = jnp.dot(ef0_ref[...], aw0_ref[...], preferred_element_type=jnp.float32)
    d = FEATURE_DIM
    efo_ref[...] = ef2 + _swish(mm[:, :d]) * jax.nn.sigmoid(mm[:, d : 2 * d]) * ew
    amsg_ref[...] = (
        _swish(mm[:, 2 * d : 3 * d]) * jax.nn.sigmoid(mm[:, 3 * d :]) * aw
    )


def _tc_edge_block(ai, aj, ef, eagg, ef0, wcat, tbw, tbb, ew0, aw0):
    grid = (N_EDGES // _EB_BLK,)
    d = FEATURE_DIM
    return pl.pallas_call(
        _tc_edge_block_body,
        grid=grid,
        in_specs=[
            pl.BlockSpec((_EB_BLK, d), lambda i: (i, 0)),
            pl.BlockSpec((_EB_BLK, d), lambda i: (i, 0)),
            pl.BlockSpec((_EB_BLK, d), lambda i: (i, 0)),
            pl.BlockSpec((_EB_BLK, ANGLE_DIM), lambda i: (i, 0)),
            pl.BlockSpec((_EB_BLK, 16), lambda i: (i, 0)),
            pl.BlockSpec((3 * d, 4 * d), lambda i: (0, 0)),
            pl.BlockSpec((ANGLE_DIM, d), lambda i: (0, 0)),
            pl.BlockSpec((1, d), lambda i: (0, 0)),
            pl.BlockSpec((16, d), lambda i: (0, 0)),
            pl.BlockSpec((16, d), lambda i: (0, 0)),
        ],
        out_specs=[
            pl.BlockSpec((_EB_BLK, d), lambda i: (i, 0)),
            pl.BlockSpec((_EB_BLK, d), lambda i: (i, 0)),
        ],
        out_shape=[
            jax.ShapeDtypeStruct((N_EDGES, d), jnp.float32),
            jax.ShapeDtypeStruct((N_EDGES, d), jnp.float32),
        ],
    )(ai, aj, ef, eagg, ef0, wcat, tbw, tbb, ew0, aw0)


_ATB = 2000


def _tc_add3_body(a_ref, p0_ref, p1_ref, out_ref):
    out_ref[...] = a_ref[...] + p0_ref[...] + p1_ref[...]


def _tc_add3(a, p0, p1):
    grid = (N_ATOMS // _ATB,)
    d = FEATURE_DIM
    spec = pl.BlockSpec((_ATB, d), lambda i: (i, 0))
    return pl.pallas_call(
        _tc_add3_body,
        grid=grid,
        in_specs=[spec, spec, spec],
        out_specs=spec,
        out_shape=jax.ShapeDtypeStruct((N_ATOMS, d), jnp.float32),
    )(a, p0, p1)


def _tc_final_body(af_ref, w1_ref, b1_ref, w2_ref, b2_ref, w3r_ref, b3_ref,
                   out_ref):
    h = _swish(
        jnp.dot(af_ref[0], w1_ref[...], preferred_element_type=jnp.float32)
        + b1_ref[...]
    )
    h = _swish(
        jnp.dot(h, w2_ref[...], preferred_element_type=jnp.float32)
        + b2_ref[...]
    )
    s = jnp.sum(h * w3r_ref[...]) + ATOMS_PER * b3_ref[0, 0]
    out_ref[...] = jnp.broadcast_to(s, (1, 1, FEATURE_DIM))


def _tc_final(atom_f, w1, b1, w2, b2, w3r, b3):
    d = FEATURE_DIM
    af3 = atom_f.reshape(B, ATOMS_PER, d)
    return pl.pallas_call(
        _tc_final_body,
        grid=(B,),
        in_specs=[
            pl.BlockSpec((1, ATOMS_PER, d), lambda i: (i, 0, 0)),
            pl.BlockSpec((d, d), lambda i: (0, 0)),
            pl.BlockSpec((1, d), lambda i: (0, 0)),
            pl.BlockSpec((d, d), lambda i: (0, 0)),
            pl.BlockSpec((1, d), lambda i: (0, 0)),
            pl.BlockSpec((1, d), lambda i: (0, 0)),
            pl.BlockSpec(memory_space=pltpu.SMEM),
        ],
        out_specs=pl.BlockSpec((1, 1, FEATURE_DIM), lambda i: (i, 0, 0)),
        out_shape=jax.ShapeDtypeStruct((B, 1, FEATURE_DIM), jnp.float32),
    )(af3, w1, b1, w2, b2, w3r, b3)


def kernel(atomic_numbers, pos, edge_index, edge_offsets, cell, three_body_indices, total_num_atoms, total_num_edges, total_num_angles, embed_table, enc_W, enc_b, tb_gate_W_0, tb_edge_W_0, tb_edge_b_0, e_phi_W_0, e_sig_W_0, e_w0_0, a_phi_W_0, a_sig_W_0, a_w0_0, tb_gate_W_1, tb_edge_W_1, tb_edge_b_1, e_phi_W_1, e_sig_W_1, e_w0_1, a_phi_W_1, a_sig_W_1, a_w0_1, tb_gate_W_2, tb_edge_W_2, tb_edge_b_2, e_phi_W_2, e_sig_W_2, e_w0_2, a_phi_W_2, a_sig_W_2, a_w0_2, tb_gate_W_3, tb_edge_W_3, tb_edge_b_3, e_phi_W_3, e_sig_W_3, e_w0_3, a_phi_W_3, a_sig_W_3, a_w0_3, en_W1, en_b1, en_W2, en_b2, en_W3, en_b3):
    blocks = [
        (tb_gate_W_0, tb_edge_W_0, tb_edge_b_0, e_phi_W_0, e_sig_W_0, e_w0_0, a_phi_W_0, a_sig_W_0, a_w0_0),
        (tb_gate_W_1, tb_edge_W_1, tb_edge_b_1, e_phi_W_1, e_sig_W_1, e_w0_1, a_phi_W_1, a_sig_W_1, a_w0_1),
        (tb_gate_W_2, tb_edge_W_2, tb_edge_b_2, e_phi_W_2, e_sig_W_2, e_w0_2, a_phi_W_2, a_sig_W_2, a_w0_2),
        (tb_gate_W_3, tb_edge_W_3, tb_edge_b_3, e_phi_W_3, e_sig_W_3, e_w0_3, a_phi_W_3, a_sig_W_3, a_w0_3),
    ]

    src = edge_index[0].astype(jnp.int32)
    dst = edge_index[1].astype(jnp.int32)
    ab = (jnp.arange(N_ANGLES, dtype=jnp.int32) // ANGLES_PER)
    tbi = three_body_indices.astype(jnp.int32)
    ij_glob = tbi[:, 0] + ab * EDGES_PER
    ik_glob = tbi[:, 1] + ab * EDGES_PER
    ij_loc = tbi[:, 0] + (ab % (B // 2)) * EDGES_PER
    pos16 = jnp.pad(pos, ((0, 0), (0, 13)))
    cell9 = cell.reshape(B, 9)
    an_pad = jnp.pad(atomic_numbers.astype(jnp.int32), (0, 240))
    zeros16 = jnp.zeros((EDGES_PER * (B // 2) // 16, ANGLE_DIM), jnp.float32)
    zeros128 = jnp.zeros((N_ATOMS // 16, FEATURE_DIM), jnp.float32)
    enc_b2 = enc_b.reshape(1, FEATURE_DIM)
    eoffp = jnp.pad(edge_offsets, ((0, 0), (0, 13))).reshape(_NEP, 128)
    encw16 = jnp.pad(enc_W, ((0, 12), (0, 0)))
    encbig = jnp.kron(jnp.eye(_PACK, dtype=jnp.float32), encw16)
    encb8 = jnp.tile(enc_b.reshape(1, FEATURE_DIM), (1, _PACK))

    psd = _sc_gather(pos16, jnp.concatenate([src, dst]))
    psp = psd[:N_EDGES].reshape(_NEP, 128)
    pdp = psd[N_EDGES:].reshape(_NEP, 128)
    egeop, ef0p = _tc_edge_geom(psp, pdp, eoffp, cell9)
    edge_f = _tc_edge_enc(ef0p, encbig, encb8).reshape(N_EDGES, FEATURE_DIM)
    egeo = egeop.reshape(N_EDGES, 16)
    ef0 = ef0p.reshape(N_EDGES, 16)
    gcat = _sc_gather(egeo, jnp.concatenate([ij_glob, ik_glob]))
    gijp = gcat[:N_ANGLES].reshape(_NAP, 128)
    gikp = gcat[N_ANGLES:].reshape(_NAP, 128)
    wp = _tc_angle_geom(gijp, gikp)

    atom_f = _sc_gather(embed_table, an_pad)[:N_ATOMS]

    dstsrc = jnp.concatenate([dst, src])
    dstik = _sc_dstik(dst, tbi[:, 1])
    for (gate_W, tbw, tbb, e_phi, e_sig, e_w0, a_phi, a_sig, a_w0) in blocks:
        wcat = jnp.concatenate([e_phi, e_sig, a_phi, a_sig], axis=1)
        ajai = _sc_gather(atom_f, dstsrc)
        aj = ajai[:N_EDGES]
        ai = ajai[N_EDGES:]
        gate_atom = _tc_gate(atom_f, gate_W)
        gate_ik = _sc_gather(gate_atom, dstik)
        msg = _tc_msg(wp, gate_ik.reshape(_NAP, 128)).reshape(N_ANGLES, 16)
        eagg = _sc_scatter_add(msg, ij_loc, zeros16, EDGES_PER * (B // 2))
        eagg = eagg.reshape(N_EDGES, ANGLE_DIM)
        edge_f, amsg = _tc_edge_block(
            ai, aj, edge_f, eagg, ef0, wcat, tbw,
            tbb.reshape(1, FEATURE_DIM),
            jnp.pad(e_w0, ((0, 12), (0, 0))),
            jnp.pad(a_w0, ((0, 12), (0, 0))),
        )
        parts = _sc_scatter_add(amsg, src, zeros128, N_ATOMS)
        atom_f = _tc_add3(atom_f, parts[0], parts[1])

    out = _tc_final(
        atom_f, en_W1, en_b1.reshape(1, FEATURE_DIM),
        en_W2, en_b2.reshape(1, FEATURE_DIM),
        en_W3.reshape(1, FEATURE_DIM), en_b3.reshape(1, 1),
    )
    return out[:, 0, 0]
